# baseline (speedup 1.0000x reference)
import jax
import jax.numpy as jnp
from jax.experimental import pallas as pl
from jax.experimental.pallas import tpu as pltpu

_D = 256
_T = 3072
_CHUNK = 256
_NCHUNK = _T // _CHUNK
_CHUNK_LEVELS = 3
_POUT = _CHUNK >> _CHUNK_LEVELS
_PARTS = _NCHUNK * _POUT


def _lstm_merge(hcat, ccat, W, b):
    d = _D
    g = jnp.dot(hcat, W, preferred_element_type=jnp.float32) + b
    i = jax.nn.sigmoid(g[:, 0 * d:1 * d])
    o = jax.nn.sigmoid(g[:, 1 * d:2 * d])
    u = jnp.tanh(g[:, 2 * d:3 * d])
    fl = jax.nn.sigmoid(g[:, 3 * d:4 * d])
    fr = jax.nn.sigmoid(g[:, 4 * d:5 * d])
    c = i * u + fl * ccat[:, :d] + fr * ccat[:, d:]
    h = o * jnp.tanh(c)
    return h, c


def _reduce_levels(h, c, W, b, stop_at):
    n = h.shape[0]
    while n > stop_at:
        m = n // 2
        h, c = _lstm_merge(h.reshape(m, 2 * _D), c.reshape(m, 2 * _D), W, b)
        n = m
    return h, c


def _fenwick_kernel(h_ref, c_ref, Wm_ref, bm_ref, Ws_ref, bs_ref,
                    ho_ref, co_ref, hs_ref, cs_ref):
    step = pl.program_id(0)
    Wm = Wm_ref[...]
    bm = bm_ref[0]

    @pl.when(step < _NCHUNK)
    def _chunk():
        h, c = _reduce_levels(h_ref[...], c_ref[...], Wm, bm, _POUT)
        hs_ref[pl.ds(step * _POUT, _POUT), :] = h
        cs_ref[pl.ds(step * _POUT, _POUT), :] = c

    @pl.when(step == _NCHUNK)
    def _tail():
        h, c = _reduce_levels(hs_ref[...], cs_ref[...], Wm, bm, 3)
        hA, cA = _lstm_merge(h[0:2].reshape(1, 2 * _D),
                             c[0:2].reshape(1, 2 * _D), Wm, bm)
        hf, cf = _lstm_merge(jnp.concatenate([h[2:3], hA], axis=1),
                             jnp.concatenate([c[2:3], cA], axis=1),
                             Ws_ref[...], bs_ref[0])
        ho_ref[...] = hf
        co_ref[...] = cf


def kernel(states_h, states_c, W_merge, b_merge, W_sum, b_sum):
    out_shape = (jax.ShapeDtypeStruct((1, _D), jnp.float32),
                 jax.ShapeDtypeStruct((1, _D), jnp.float32))
    chunk_map = lambda i: (jnp.minimum(i, _NCHUNK - 1), 0)
    fixed = lambda i: (0, 0)
    h, c = pl.pallas_call(
        _fenwick_kernel,
        grid=(_NCHUNK + 1,),
        in_specs=[
            pl.BlockSpec((_CHUNK, _D), chunk_map),
            pl.BlockSpec((_CHUNK, _D), chunk_map),
            pl.BlockSpec((2 * _D, 5 * _D), fixed),
            pl.BlockSpec((1, 5 * _D), fixed),
            pl.BlockSpec((2 * _D, 5 * _D), fixed),
            pl.BlockSpec((1, 5 * _D), fixed),
        ],
        out_specs=(pl.BlockSpec((1, _D), fixed),
                   pl.BlockSpec((1, _D), fixed)),
        out_shape=out_shape,
        scratch_shapes=[pltpu.VMEM((_PARTS, _D), jnp.float32),
                        pltpu.VMEM((_PARTS, _D), jnp.float32)],
    )(states_h, states_c, W_merge, b_merge.reshape(1, -1),
      W_sum, b_sum.reshape(1, -1))
    return (h, c)


# speedup vs baseline: 1.5560x; 1.5560x over previous
import jax
import jax.numpy as jnp
from jax.experimental import pallas as pl
from jax.experimental.pallas import tpu as pltpu

_D = 256
_T = 3072


def _lstm_merge(hcat, ccat, W, b):
    d = _D
    g = jnp.dot(hcat, W, preferred_element_type=jnp.float32) + b
    i = jax.nn.sigmoid(g[:, 0 * d:1 * d])
    o = jax.nn.sigmoid(g[:, 1 * d:2 * d])
    u = jnp.tanh(g[:, 2 * d:3 * d])
    fl = jax.nn.sigmoid(g[:, 3 * d:4 * d])
    fr = jax.nn.sigmoid(g[:, 4 * d:5 * d])
    c = i * u + fl * ccat[:, :d] + fr * ccat[:, d:]
    h = o * jnp.tanh(c)
    return h, c


def _fenwick_kernel(h_ref, c_ref, Wm_ref, bm_ref, Ws_ref, bs_ref,
                    ho_ref, co_ref):
    h = h_ref[...]
    c = c_ref[...]
    Wm = Wm_ref[...]
    bm = bm_ref[0]
    Ws = Ws_ref[...]
    bs = bs_ref[0]

    n = _T
    while n > 3:
        m = n // 2
        h, c = _lstm_merge(h.reshape(m, 2 * _D), c.reshape(m, 2 * _D),
                           Wm, bm)
        n = m

    hA, cA = _lstm_merge(h[0:2].reshape(1, 2 * _D),
                         c[0:2].reshape(1, 2 * _D), Wm, bm)
    hB = h[2:3]
    cB = c[2:3]
    hf, cf = _lstm_merge(jnp.concatenate([hB, hA], axis=1),
                         jnp.concatenate([cB, cA], axis=1), Ws, bs)
    ho_ref[...] = hf
    co_ref[...] = cf


def kernel(states_h, states_c, W_merge, b_merge, W_sum, b_sum):
    out_shape = (jax.ShapeDtypeStruct((1, _D), jnp.float32),
                 jax.ShapeDtypeStruct((1, _D), jnp.float32))
    h, c = pl.pallas_call(
        _fenwick_kernel,
        out_shape=out_shape,
    )(states_h, states_c, W_merge, b_merge.reshape(1, -1),
      W_sum, b_sum.reshape(1, -1))
    return (h, c)
